# baseline (speedup 1.0000x reference)
import jax
import jax.numpy as jnp
from jax.experimental import pallas as pl

N_NODE = 10000
KSIZE = 3


def _tconv(X, w1, b1, w2, b2, w3, b3):
    Xp = jnp.transpose(X, (0, 3, 2, 1))

    def conv(w, b):
        y = jax.lax.conv_general_dilated(Xp, w, window_strides=(1, 1), padding='VALID')
        return y + b[None, :, None, None]

    P = conv(w1, b1)
    Q = jax.nn.sigmoid(conv(w2, b2))
    H = jax.nn.relu(P * Q + conv(w3, b3))
    return jnp.transpose(H, (0, 3, 2, 1))


def _cheb(x, edge_index, edge_weight, W0, W1, cb):
    row = edge_index[0]
    col = edge_index[1]
    deg = jax.ops.segment_sum(edge_weight, row, num_segments=N_NODE)
    dinv = jnp.where(deg > 0, deg ** -0.5, 0.0)
    norm = -(dinv[row] * edge_weight * dinv[col])
    xn = jnp.moveaxis(x, 2, 0)
    msg = xn[row] * norm[:, None, None, None]
    agg = jax.ops.segment_sum(msg, col, num_segments=N_NODE)
    Tx1 = jnp.moveaxis(agg, 0, 2)
    return x @ W0 + Tx1 @ W1 + cb


def _tail_kernel(t_ref, w_ref, b_ref, o_ref):
    t = t_ref[...]
    nb, B, T2, C = t.shape
    f = t.reshape(nb, B * T2 * C)
    mean = jnp.mean(f, axis=1, keepdims=True)
    var = jnp.mean(f * f, axis=1, keepdims=True) - mean * mean
    h = jax.nn.relu((f - mean) * jax.lax.rsqrt(var + 1e-5))
    h = h.reshape(nb, B, T2 * C)
    for b in range(B):
        o_ref[:, b, :] = h[:, b, :] @ w_ref[...] + b_ref[...]


def kernel(x, edge_index, edge_weight, tc1_w1, tc1_b1, tc1_w2, tc1_b2, tc1_w3, tc1_b3,
           cheb_W0, cheb_W1, cheb_b, tc2_w1, tc2_b1, tc2_w2, tc2_b2, tc2_w3, tc2_b3,
           lin_w, lin_b):
    T0 = _tconv(x, tc1_w1, tc1_b1, tc1_w2, tc1_b2, tc1_w3, tc1_b3)
    T = _cheb(T0, edge_index, edge_weight, cheb_W0, cheb_W1, cheb_b)
    T = jax.nn.relu(T)
    T = _tconv(T, tc2_w1, tc2_b1, tc2_w2, tc2_b2, tc2_w3, tc2_b3)
    Tn = jnp.transpose(T, (2, 0, 1, 3))
    B, T2, N, C = T.shape[0], T.shape[1], T.shape[2], T.shape[3]
    NB = 1000
    out = pl.pallas_call(
        _tail_kernel,
        grid=(N // NB,),
        in_specs=[
            pl.BlockSpec((NB, B, T2, C), lambda i: (i, 0, 0, 0)),
            pl.BlockSpec((T2 * C, lin_w.shape[1]), lambda i: (0, 0)),
            pl.BlockSpec((lin_w.shape[1],), lambda i: (0,)),
        ],
        out_specs=pl.BlockSpec((NB, B, lin_w.shape[1]), lambda i: (i, 0, 0)),
        out_shape=jax.ShapeDtypeStruct((N, B, lin_w.shape[1]), jnp.float32),
    )(Tn, lin_w, lin_b)
    return jnp.transpose(out, (1, 0, 2))


# speedup vs baseline: 23.9909x; 23.9909x over previous
import jax
import jax.numpy as jnp
from jax import lax
from jax.experimental import pallas as pl
from jax.experimental.pallas import tpu as pltpu
from jax.experimental.pallas import tpu_sc as plsc

N = 10000
C1 = 16
CH = 32
DD = 7
KS = 3
TE = 8
T1 = 6
T2 = 4
B = 2
E = 160000
S = B * T1
G = 3
GW = 4 * CH
EP = 163840
PT = EP // 32
NTILE = 16
CHUNK = 128
EPH = EP // 2
ETJ0 = EP // NTILE
ETJ1 = EPH // NTILE
ET = ETJ0 + ETJ1
NCHA = ETJ0 // CHUNK
NCHB = ETJ1 // CHUNK
NT = ET // CHUNK
AF = N // CHUNK
AKF = AF // NTILE
AREM = AF - AKF * NTILE
ATAIL = N - AF * CHUNK


def _deg_body(row_hbm, w_hbm, out_hbm, row_v, w_v, acc_v):
    cid = lax.axis_index("c")
    sid = lax.axis_index("s")
    wid = sid * 2 + cid

    def zbody(i, _):
        acc_v[pl.ds(i * 16, 16)] = jnp.zeros((16,), jnp.float32)
        return 0

    lax.fori_loop(0, N // 16, zbody, 0)
    pltpu.sync_copy(row_hbm.at[pl.ds(wid * PT, PT)], row_v)
    pltpu.sync_copy(w_hbm.at[pl.ds(wid * PT, PT)], w_v)

    def ebody(i, _):
        idx = row_v[pl.ds(i * 16, 16)]
        vals = w_v[pl.ds(i * 16, 16)]
        plsc.addupdate_scatter(acc_v, [idx], vals)
        return 0

    lax.fori_loop(0, PT // 16, ebody, 0)
    pltpu.sync_copy(acc_v, out_hbm.at[wid])


_deg_call = pl.kernel(
    _deg_body,
    out_type=jax.ShapeDtypeStruct((32, N), jnp.float32),
    mesh=plsc.VectorSubcoreMesh(core_axis_name="c", subcore_axis_name="s",
                                num_cores=2, num_subcores=16),
    compiler_params=pltpu.CompilerParams(needs_layout_passes=False),
    scratch_types=[
        pltpu.VMEM((PT,), jnp.int32),
        pltpu.VMEM((PT,), jnp.float32),
        pltpu.VMEM((N,), jnp.float32),
    ],
)


def _dinv_body(p_ref, o_ref):
    deg = jnp.sum(p_ref[...], axis=0)
    o_ref[...] = jnp.where(deg > 0.0, lax.rsqrt(deg), 0.0)


def _dinv_call(parts):
    return pl.pallas_call(
        _dinv_body,
        out_shape=jax.ShapeDtypeStruct((N,), jnp.float32),
    )(parts)


def _tc1_body(x_ref, w1_ref, b1_ref, w2_ref, b2_ref, w3_ref, b3_ref, o_ref):
    for b in range(B):
        for t in range(T1):
            s = b * T1 + t
            xc = jnp.concatenate([x_ref[b, t + k] for k in range(KS)], axis=1)
            p = jnp.dot(xc, w1_ref[...], preferred_element_type=jnp.float32) + b1_ref[...]
            q = jnp.dot(xc, w2_ref[...], preferred_element_type=jnp.float32) + b2_ref[...]
            r = jnp.dot(xc, w3_ref[...], preferred_element_type=jnp.float32) + b3_ref[...]
            o_ref[s // 4, :, s % 4, :] = jnp.maximum(p * jax.nn.sigmoid(q) + r, 0.0)


def _tc1_call(x, w1c, b1, w2c, b2, w3c, b3):
    nb = 1000
    grid = N // nb
    return pl.pallas_call(
        _tc1_body,
        grid=(grid,),
        in_specs=[
            pl.BlockSpec((B, TE, nb, C1), lambda i: (0, 0, i, 0)),
            pl.BlockSpec((KS * C1, CH), lambda i: (0, 0)),
            pl.BlockSpec((CH,), lambda i: (0,)),
            pl.BlockSpec((KS * C1, CH), lambda i: (0, 0)),
            pl.BlockSpec((CH,), lambda i: (0,)),
            pl.BlockSpec((KS * C1, CH), lambda i: (0, 0)),
            pl.BlockSpec((CH,), lambda i: (0,)),
        ],
        out_specs=pl.BlockSpec((G, nb, 4, CH), lambda i: (0, i, 0, 0)),
        out_shape=jax.ShapeDtypeStruct((G, N, 4, CH), jnp.float32),
    )(x, w1c, b1, w2c, b2, w3c, b3)


BLK = 1024
BLKR = BLK // CHUNK
NBT0 = ETJ0 // BLK
NBT1 = ETJ1 // BLK
NBT = NBT0 + NBT1


def _sw_body(row_hbm, col_hbm, w_hbm, dinv_hbm, s_hbm, row_v, col_v, w_v,
             dinv_v):
    cid = lax.axis_index("c")
    sid = lax.axis_index("s")
    wid = sid * 2 + cid
    pltpu.sync_copy(row_hbm.at[pl.ds(wid * PT, PT)], row_v)
    pltpu.sync_copy(col_hbm.at[pl.ds(wid * PT, PT)], col_v)
    pltpu.sync_copy(w_hbm.at[pl.ds(wid * PT, PT)], w_v)
    pltpu.sync_copy(dinv_hbm, dinv_v)

    def sbody(i, _):
        off = i * 16
        r = row_v[pl.ds(off, 16)]
        c = col_v[pl.ds(off, 16)]
        dr = plsc.load_gather(dinv_v, [r])
        dc = plsc.load_gather(dinv_v, [c])
        w_v[pl.ds(off, 16)] = -(dr * w_v[pl.ds(off, 16)] * dc)
        return 0

    lax.fori_loop(0, PT // 16, sbody, 0)
    pltpu.sync_copy(w_v, s_hbm.at[pl.ds(wid * PT, PT)])


_sw_call = pl.kernel(
    _sw_body,
    out_type=jax.ShapeDtypeStruct((EP,), jnp.float32),
    mesh=plsc.VectorSubcoreMesh(core_axis_name="c", subcore_axis_name="s",
                                num_cores=2, num_subcores=16),
    compiler_params=pltpu.CompilerParams(needs_layout_passes=False),
    scratch_types=[
        pltpu.VMEM((PT,), jnp.int32),
        pltpu.VMEM((PT,), jnp.int32),
        pltpu.VMEM((PT,), jnp.float32),
        pltpu.VMEM((N,), jnp.float32),
    ],
)


def _agg_body(t0_hbm, row_hbm, colg_hbm, s_hbm, out_hbm,
              row_b, col_b, s_b, gbuf, acc_sh, sem_g, sem_e):
    cid = lax.axis_index("c")
    sid = lax.axis_index("s")

    def _edge_offsets(bt):
        e_off = jnp.where(bt < NBT0, sid * ETJ0 + bt * BLK,
                          cid * EPH + sid * ETJ1 + (bt - NBT0) * BLK)
        c_off = jnp.where(bt < NBT0, sid * (ETJ0 // CHUNK) + bt * BLKR,
                          cid * (EPH // CHUNK) + sid * (ETJ1 // CHUNK)
                          + (bt - NBT0) * BLKR)
        return e_off, c_off

    def _start_edge_dma(bt, eb):
        e_off, c_off = _edge_offsets(bt)
        pltpu.make_async_copy(row_hbm.at[pl.ds(e_off, BLK)],
                              row_b.at[eb], sem_e).start()
        pltpu.make_async_copy(colg_hbm.at[pl.ds(c_off, BLKR)],
                              col_b.at[eb], sem_e).start()
        pltpu.make_async_copy(s_hbm.at[pl.ds(e_off, BLK)],
                              s_b.at[eb], sem_e).start()

    def _wait_edge_dma(eb):
        pltpu.make_async_copy(row_hbm.at[pl.ds(0, BLK)],
                              row_b.at[eb], sem_e).wait()
        pltpu.make_async_copy(colg_hbm.at[pl.ds(0, BLKR)],
                              col_b.at[eb], sem_e).wait()
        pltpu.make_async_copy(s_hbm.at[pl.ds(0, BLK)],
                              s_b.at[eb], sem_e).wait()

    def _zero_gbuf0():
        def zm(i, _):
            for q in range(GW // 16):
                gbuf[0, i, pl.ds(q * 16, 16)] = jnp.zeros((16,), jnp.float32)
            return 0

        lax.fori_loop(0, CHUNK, zm, 0)

    def _zero_acc():
        _zero_gbuf0()

        def za(k, _):
            off = (k * NTILE + sid) * CHUNK
            pltpu.sync_copy(gbuf.at[0], acc_sh.at[pl.ds(off, CHUNK)])
            return 0

        lax.fori_loop(0, AKF, za, 0)

        @pl.when(sid < AREM)
        def _zr():
            off = (AKF * NTILE + sid) * CHUNK
            pltpu.sync_copy(gbuf.at[0], acc_sh.at[pl.ds(off, CHUNK)])

        @pl.when(sid == AREM)
        def _zt():
            pltpu.sync_copy(gbuf.at[0, pl.ds(0, ATAIL)],
                            acc_sh.at[pl.ds(AF * CHUNK, ATAIL)])

    def _dump_acc(base):
        def da(k, _):
            off = (k * NTILE + sid) * CHUNK
            pltpu.sync_copy(acc_sh.at[pl.ds(off, CHUNK)],
                            out_hbm.at[pl.ds(base + off, CHUNK)])
            return 0

        lax.fori_loop(0, AKF, da, 0)

        @pl.when(sid < AREM)
        def _dr():
            off = (AKF * NTILE + sid) * CHUNK
            pltpu.sync_copy(acc_sh.at[pl.ds(off, CHUNK)],
                            out_hbm.at[pl.ds(base + off, CHUNK)])

        @pl.when(sid == AREM)
        def _dt():
            pltpu.sync_copy(acc_sh.at[pl.ds(AF * CHUNK, ATAIL)],
                            out_hbm.at[pl.ds(base + AF * CHUNK, ATAIL)])

    _zero_acc()
    plsc.subcore_barrier()
    _start_edge_dma(0, 0)

    def _do_block(bt, eb):
        _wait_edge_dma(eb)

        @pl.when(bt + 1 < NBT)
        def _pf():
            _start_edge_dma(bt + 1, 1 - eb)

        base = jnp.where(bt < NBT0, cid * N, 2 * N)

        def rb(i, _):
            row_b[eb, pl.ds(i * 16, 16)] = (
                row_b[eb, pl.ds(i * 16, 16)] + base)
            return 0

        lax.fori_loop(0, BLK // 16, rb, 0)

        pltpu.make_async_copy(
            t0_hbm.at[row_b.at[eb, pl.ds(0, CHUNK)]], gbuf.at[0],
            sem_g).start()

        def cpair(cp, _):
            for cc in range(2):
                c = cp * 2 + cc
                pltpu.make_async_copy(
                    t0_hbm.at[row_b.at[eb, pl.ds(0, CHUNK)]], gbuf.at[cc],
                    sem_g).wait()

                @pl.when(c + 1 < BLKR)
                def _go():
                    pltpu.make_async_copy(
                        t0_hbm.at[row_b.at[eb, pl.ds((c + 1) * CHUNK, CHUNK)]],
                        gbuf.at[1 - cc], sem_g).start()

                def mul16(g16, _):
                    sv16 = s_b[eb, pl.ds(c * CHUNK + g16 * 16, 16)]
                    for e in range(16):
                        ee = g16 * 16 + e
                        sv = sv16[e]
                        for q in range(GW // 16):
                            gbuf[cc, ee, pl.ds(q * 16, 16)] = (
                                gbuf[cc, ee, pl.ds(q * 16, 16)] * sv)
                    return 0

                lax.fori_loop(0, CHUNK // 16, mul16, 0)
                pltpu.sync_copy(gbuf.at[cc], acc_sh.at[col_b.at[eb, c]],
                                add=True)
            return 0

        lax.fori_loop(0, BLKR // 2, cpair, 0)

        @pl.when(bt == NBT0 - 1)
        def _trans():
            plsc.subcore_barrier()
            _dump_acc(cid * N)
            plsc.subcore_barrier()
            _zero_acc()
            plsc.subcore_barrier()

    def bpair(bp, _):
        for i in range(2):
            _do_block(bp * 2 + i, i)
        return 0

    lax.fori_loop(0, (NBT - 1) // 2, bpair, 0)
    _do_block(NBT - 1, 0)
    plsc.subcore_barrier()
    _dump_acc((2 + cid) * N)


_agg_call = pl.kernel(
    _agg_body,
    out_type=jax.ShapeDtypeStruct((4 * N, GW), jnp.float32),
    mesh=plsc.VectorSubcoreMesh(core_axis_name="c", subcore_axis_name="s",
                                num_cores=2, num_subcores=16),
    compiler_params=pltpu.CompilerParams(needs_layout_passes=False),
    scratch_types=[
        pltpu.VMEM((2, BLK), jnp.int32),
        pltpu.VMEM((2, BLKR, CHUNK), jnp.int32),
        pltpu.VMEM((2, BLK), jnp.float32),
        pltpu.VMEM((2, CHUNK, GW), jnp.float32),
        pltpu.VMEM_SHARED((N, GW), jnp.float32),
        pltpu.SemaphoreType.DMA,
        pltpu.SemaphoreType.DMA,
    ],
)


def _post_body(t0_ref, ag_ref, W0_ref, W1_ref, cb_ref,
               w1_ref, b1_ref, w2_ref, b2_ref, w3_ref, b3_ref,
               lw_ref, lb_ref, o_ref):
    nb = t0_ref.shape[1]
    t0_list, ag_list = [], []
    for s in range(S):
        g, j = s // 4, s % 4
        t0_list.append(t0_ref[g, :, j, :])
        a = ag_ref[g, :, j, :]
        if g == 2:
            a = a + ag_ref[3, :, j, :]
        ag_list.append(a)
    t0 = jnp.concatenate(t0_list, axis=0)
    ag = jnp.concatenate(ag_list, axis=0)
    cmb = (jnp.dot(t0, W0_ref[...], preferred_element_type=jnp.float32)
           + jnp.dot(ag, W1_ref[...], preferred_element_type=jnp.float32)
           + cb_ref[...])
    cmb = jnp.maximum(cmb, 0.0).reshape(B, T1, nb, CH)
    hs = []
    for b in range(B):
        for t in range(T2):
            xc = jnp.concatenate([cmb[b, t + k] for k in range(KS)], axis=1)
            p = jnp.dot(xc, w1_ref[...], preferred_element_type=jnp.float32) + b1_ref[...]
            q = jnp.dot(xc, w2_ref[...], preferred_element_type=jnp.float32) + b2_ref[...]
            r = jnp.dot(xc, w3_ref[...], preferred_element_type=jnp.float32) + b3_ref[...]
            hs.append(jnp.maximum(p * jax.nn.sigmoid(q) + r, 0.0))
    ssum = sum(jnp.sum(h, axis=1) for h in hs)
    ssq = sum(jnp.sum(h * h, axis=1) for h in hs)
    cnt = float(B * T2 * CH)
    mean = ssum / cnt
    var = ssq / cnt - mean * mean
    sc = lax.rsqrt(var + 1e-5)
    for b in range(B):
        acc = jnp.zeros((nb, DD), jnp.float32) + lb_ref[...]
        for t in range(T2):
            h = hs[b * T2 + t]
            hn = jnp.maximum((h - mean[:, None]) * sc[:, None], 0.0)
            acc = acc + jnp.dot(hn, lw_ref[t], preferred_element_type=jnp.float32)
        o_ref[b] = acc


def _post_call(t0p, ag, W0, W1, cb, w1c, b1, w2c, b2, w3c, b3, lw, lb):
    nb = 1000
    grid = N // nb
    return pl.pallas_call(
        _post_body,
        grid=(grid,),
        in_specs=[
            pl.BlockSpec((G, nb, 4, CH), lambda i: (0, i, 0, 0)),
            pl.BlockSpec((4, nb, 4, CH), lambda i: (0, i, 0, 0)),
            pl.BlockSpec((CH, CH), lambda i: (0, 0)),
            pl.BlockSpec((CH, CH), lambda i: (0, 0)),
            pl.BlockSpec((CH,), lambda i: (0,)),
            pl.BlockSpec((KS * CH, CH), lambda i: (0, 0)),
            pl.BlockSpec((CH,), lambda i: (0,)),
            pl.BlockSpec((KS * CH, CH), lambda i: (0, 0)),
            pl.BlockSpec((CH,), lambda i: (0,)),
            pl.BlockSpec((KS * CH, CH), lambda i: (0, 0)),
            pl.BlockSpec((CH,), lambda i: (0,)),
            pl.BlockSpec((T2, CH, DD), lambda i: (0, 0, 0)),
            pl.BlockSpec((DD,), lambda i: (0,)),
        ],
        out_specs=pl.BlockSpec((B, nb, DD), lambda i: (0, i, 0)),
        out_shape=jax.ShapeDtypeStruct((B, N, DD), jnp.float32),
    )(t0p, ag, W0, W1, cb, w1c, b1, w2c, b2, w3c, b3, lw, lb)


def _cat_w(w):
    return jnp.concatenate([w[:, :, 0, k].T for k in range(KS)], axis=0)


def kernel(x, edge_index, edge_weight, tc1_w1, tc1_b1, tc1_w2, tc1_b2,
           tc1_w3, tc1_b3, cheb_W0, cheb_W1, cheb_b, tc2_w1, tc2_b1,
           tc2_w2, tc2_b2, tc2_w3, tc2_b3, lin_w, lin_b):
    row = edge_index[0].astype(jnp.int32)
    col = edge_index[1].astype(jnp.int32)
    padn = EP - E
    rowp = jnp.concatenate([row, jnp.zeros((padn,), jnp.int32)])
    colp = jnp.concatenate([col, jnp.zeros((padn,), jnp.int32)])
    wp = jnp.concatenate([edge_weight.astype(jnp.float32),
                          jnp.zeros((padn,), jnp.float32)])
    colg = colp.reshape(EP // CHUNK, CHUNK)

    parts = _deg_call(rowp, wp)
    dinv = _dinv_call(parts)
    sp = _sw_call(rowp, colp, wp, dinv)

    t0p = _tc1_call(x, _cat_w(tc1_w1), tc1_b1, _cat_w(tc1_w2), tc1_b2,
                    _cat_w(tc1_w3), tc1_b3)
    t0pf = t0p.reshape(G * N, GW)
    aggf = _agg_call(t0pf, rowp, colg, sp)

    out = _post_call(t0p, aggf.reshape(4, N, 4, CH),
                     cheb_W0, cheb_W1, cheb_b,
                     _cat_w(tc2_w1), tc2_b1, _cat_w(tc2_w2), tc2_b2,
                     _cat_w(tc2_w3), tc2_b3,
                     lin_w.reshape(T2, CH, DD), lin_b)
    return out
